# baseline (speedup 1.0000x reference)
import functools

import jax
import jax.numpy as jnp
from jax import lax
from jax.experimental import pallas as pl
from jax.experimental.pallas import tpu as pltpu
from jax.experimental.pallas import tpu_sc as plsc

F = 26
D = 64


def _gather_alpha(alpha_attention, idx2d):
    NB, C = idx2d.shape
    B = NB * C
    NC, NS = 2, 16
    NW = NC * NS
    rows_per_w = NB // NW
    b_per_w = B // NW
    mesh = plsc.VectorSubcoreMesh(core_axis_name="c", subcore_axis_name="s")

    @functools.partial(
        pl.kernel,
        mesh=mesh,
        compiler_params=pltpu.CompilerParams(use_tc_tiling_on_sc=False),
        out_type=jax.ShapeDtypeStruct((B, F), jnp.float32),
        scratch_types=[
            pltpu.VMEM((rows_per_w, C), jnp.int32),
            pltpu.VMEM((b_per_w, F), jnp.float32),
            pltpu.SemaphoreType.DMA,
        ],
    )
    def gather_k(table_hbm, idx_hbm, out_hbm, idx_v, rows_v, sem):
        wid = lax.axis_index("s") * NC + lax.axis_index("c")
        pltpu.sync_copy(idx_hbm.at[pl.ds(wid * rows_per_w, rows_per_w)], idx_v)
        copies = [
            pltpu.async_copy(
                table_hbm.at[idx_v.at[j]], rows_v.at[pl.ds(j * C, C)], sem
            )
            for j in range(rows_per_w)
        ]
        for cp in copies:
            cp.wait()
        pltpu.sync_copy(rows_v, out_hbm.at[pl.ds(wid * b_per_w, b_per_w)])

    return gather_k(alpha_attention, idx2d)


def _merge(alpha_rows, stack_embeds, block_b=256):
    B = alpha_rows.shape[0]

    def body(alpha_ref, stack_ref, out_ref):
        a = jnp.exp(alpha_ref[...])
        s = jnp.sum(a, axis=1, keepdims=True)
        w = a / s
        x = stack_ref[...]
        out_ref[...] = jnp.sum(w[:, :, None] * x, axis=1)

    return pl.pallas_call(
        body,
        grid=(B // block_b,),
        in_specs=[
            pl.BlockSpec((block_b, F), lambda i: (i, 0)),
            pl.BlockSpec((block_b, F, D), lambda i: (i, 0, 0)),
        ],
        out_specs=pl.BlockSpec((block_b, D), lambda i: (i, 0)),
        out_shape=jax.ShapeDtypeStruct((B, D), jnp.float32),
    )(alpha_rows, stack_embeds)


def kernel(item_input, stack_embeds, alpha_attention):
    idx2d = item_input.reshape(-1, 128)
    alpha_rows = _gather_alpha(alpha_attention, idx2d)
    return _merge(alpha_rows, stack_embeds)


# speedup vs baseline: 1.4378x; 1.4378x over previous
import functools

import jax
import jax.numpy as jnp
from jax import lax
from jax.experimental import pallas as pl
from jax.experimental.pallas import tpu as pltpu
from jax.experimental.pallas import tpu_sc as plsc

F = 26
FP = 32
D = 64
W = 128


def _gather_alpha(table_w, gidx, offs):
    B = offs.shape[0]
    NC, NS = 2, 16
    NW = NC * NS
    b_per_w = B // NW
    n_chunks = b_per_w // 256
    mesh = plsc.VectorSubcoreMesh(core_axis_name="c", subcore_axis_name="s")

    @functools.partial(
        pl.kernel,
        mesh=mesh,
        compiler_params=pltpu.CompilerParams(needs_layout_passes=False),
        out_type=jax.ShapeDtypeStruct((B * FP,), jnp.float32),
        scratch_types=[
            pltpu.VMEM((4, W), jnp.int32),
            pltpu.VMEM((512, W), jnp.float32),
            pltpu.VMEM((b_per_w,), jnp.int32),
            pltpu.VMEM((b_per_w * FP,), jnp.float32),
            pltpu.SemaphoreType.DMA,
        ],
    )
    def gather_k(table_hbm, gidx_hbm, offs_hbm, out_hbm, gidx_v, buf, o_v, out_flat, sem):
        wid = lax.axis_index("s") * NC + lax.axis_index("c")
        pltpu.sync_copy(offs_hbm.at[pl.ds(wid * b_per_w, b_per_w)], o_v)
        for c in range(n_chunks):
            k = wid * n_chunks + c
            pltpu.sync_copy(gidx_hbm.at[pl.ds(k * 4, 4)], gidx_v)
            copies = [
                pltpu.async_copy(
                    table_hbm.at[gidx_v.at[j]], buf.at[pl.ds(j * W, W)], sem
                )
                for j in range(4)
            ]
            for cp in copies:
                cp.wait()

            def extract(g, _):
                o_vec = o_v[pl.ds(c * 256 + g * 16, 16)]
                r = g * 16 + lax.iota(jnp.int32, 16)
                obase = (c * 256 + r) * FP
                for f in range(F):
                    t = o_vec + f
                    row = r + ((t >> 7) << 8)
                    col = t & (W - 1)
                    v = plsc.load_gather(buf, [row, col])
                    plsc.store_scatter(out_flat, [obase + f], v)
                return 0

            lax.fori_loop(0, 16, extract, 0)
        pltpu.sync_copy(out_flat, out_hbm.at[pl.ds(wid * b_per_w * FP, b_per_w * FP)])

    return gather_k(table_w, gidx, offs).reshape(B, FP)


def _merge(alpha8, stack2d, expand, block_b=256):
    B, FD = stack2d.shape

    def body(alpha_ref, stack_ref, e_ref, out_ref):
        a = alpha_ref[...]
        lane = lax.broadcasted_iota(jnp.int32, a.shape, 1)
        e = jnp.where(lane < F, jnp.exp(a), 0.0)
        s = jnp.sum(e, axis=1, keepdims=True)
        w = e / s
        wx = jnp.dot(w, e_ref[...], preferred_element_type=jnp.float32)
        prod = wx * stack_ref[...]
        acc = prod[:, 0:128]
        for c in range(1, FD // 128):
            acc = acc + prod[:, c * 128 : (c + 1) * 128]
        out_ref[...] = acc[:, :D] + acc[:, D:]

    return pl.pallas_call(
        body,
        grid=(B // block_b,),
        in_specs=[
            pl.BlockSpec((block_b, FP), lambda i: (i, 0)),
            pl.BlockSpec((block_b, FD), lambda i: (i, 0)),
            pl.BlockSpec((FP, FD), lambda i: (0, 0)),
        ],
        out_specs=pl.BlockSpec((block_b, D), lambda i: (i, 0)),
        out_shape=jax.ShapeDtypeStruct((B, D), jnp.float32),
    )(alpha8, stack2d, expand)


def kernel(item_input, stack_embeds, alpha_attention):
    B = item_input.shape[0]
    n_rows = alpha_attention.shape[0] * F // W

    idx = item_input.reshape(-1)
    p = idx * F
    row_a = p >> 7
    row_b = jnp.minimum(row_a + 1, n_rows - 1)
    offs = (p & (W - 1)).astype(jnp.int32)
    ra = row_a.reshape(-1, 2, W)
    rb = row_b.reshape(-1, 2, W)
    gidx = jnp.concatenate([ra, rb], axis=1).reshape(-1, W)

    table_w = alpha_attention.reshape(n_rows, W)
    alpha8 = _gather_alpha(table_w, gidx, offs)

    fr = jnp.arange(FP)[:, None]
    fc = jnp.arange(F * D)[None, :] // D
    expand = (fr == fc).astype(jnp.float32)

    stack2d = stack_embeds.reshape(B, F * D)
    return _merge(alpha8, stack2d, expand)


# speedup vs baseline: 7.8607x; 5.4672x over previous
import functools

import jax
import jax.numpy as jnp
from jax import lax
from jax.experimental import pallas as pl
from jax.experimental.pallas import tpu as pltpu
from jax.experimental.pallas import tpu_sc as plsc

F = 26
FP = 32
D = 64
SLAB = 8


def _gather_alpha_t(alpha_t, idx):
    B = idx.shape[0]
    NC, NS = 2, 16
    NW = NC * NS
    b_per_w = B // NW
    CHUNK = 32
    W = 128
    mesh = plsc.VectorSubcoreMesh(core_axis_name="c", subcore_axis_name="s")

    @functools.partial(
        pl.kernel,
        mesh=mesh,
        compiler_params=pltpu.CompilerParams(needs_layout_passes=False),
        out_type=jax.ShapeDtypeStruct((FP, B), jnp.float32),
        scratch_types=[
            pltpu.VMEM((b_per_w,), jnp.int32),
            pltpu.VMEM((CHUNK * F, W), jnp.float32),
            pltpu.VMEM((FP, b_per_w), jnp.float32),
            pltpu.SemaphoreType.DMA,
        ],
    )
    def gather_k(table_hbm, idx_hbm, out_hbm, idx_v, buf, out_v, sem):
        wid = lax.axis_index("s") * NC + lax.axis_index("c")
        base = wid * b_per_w
        pltpu.sync_copy(idx_hbm.at[pl.ds(base, b_per_w)], idx_v)

        def chunk_body(c, _):
            for g in range(CHUNK // 16):
                wins = (idx_v[pl.ds(c * CHUNK + g * 16, 16)] >> 7) << 7
                for j in range(16):
                    col = pl.multiple_of(wins[j], W)
                    pltpu.make_async_copy(
                        table_hbm.at[:, pl.ds(col, W)],
                        buf.at[pl.ds((g * 16 + j) * F, F)],
                        sem,
                    ).start()
            for j in range(CHUNK):
                pltpu.make_async_copy(
                    table_hbm.at[:, pl.ds(0, W)],
                    buf.at[pl.ds(j * F, F)],
                    sem,
                ).wait()
            for g in range(CHUNK // 16):
                lane = idx_v[pl.ds(c * CHUNK + g * 16, 16)] & (W - 1)
                r = (g * 16 + lax.iota(jnp.int32, 16)) * F
                for f in range(F):
                    v = plsc.load_gather(buf, [r + f, lane])
                    plsc.store_scatter(
                        out_v,
                        [jnp.full((16,), f, jnp.int32),
                         c * CHUNK + g * 16 + lax.iota(jnp.int32, 16)],
                        v,
                    )
            return 0

        lax.fori_loop(0, b_per_w // CHUNK, chunk_body, 0)
        pltpu.sync_copy(out_v, out_hbm.at[:, pl.ds(base, b_per_w)])

    return gather_k(alpha_t, idx)


def _merge_t(alpha_t8, stack_t, block_l=512):
    B = alpha_t8.shape[1]

    def body(alpha_ref, stack_ref, out_ref):
        a = alpha_ref[...]
        row = lax.broadcasted_iota(jnp.int32, a.shape, 0)
        e = jnp.where(row < F, jnp.exp(a), 0.0)
        s = jnp.sum(e, axis=0, keepdims=True)
        w = e * (1.0 / s)
        acc = w[0:1, :] * stack_ref[0]
        for f in range(1, F):
            acc = acc + w[f : f + 1, :] * stack_ref[f]
        out_ref[...] = acc

    return pl.pallas_call(
        body,
        grid=(B // block_l,),
        in_specs=[
            pl.BlockSpec((FP, block_l), lambda i: (0, i)),
            pl.BlockSpec((F, D, block_l), lambda i: (0, 0, i)),
        ],
        out_specs=pl.BlockSpec((D, block_l), lambda i: (0, i)),
        out_shape=jax.ShapeDtypeStruct((D, B), jnp.float32),
    )(alpha_t8, stack_t)


def kernel(item_input, stack_embeds, alpha_attention):
    idx = item_input.reshape(-1)
    alpha_t = alpha_attention.T
    stack_t = jnp.transpose(stack_embeds, (1, 2, 0))
    alpha_t8 = _gather_alpha_t(alpha_t, idx)
    out_t = _merge_t(alpha_t8, stack_t)
    return out_t.T
